# baseline (speedup 1.0000x reference)
import functools

import numpy as np
import jax
import jax.numpy as jnp
from jax import lax
from jax.experimental import pallas as pl
from jax.experimental.pallas import tpu as pltpu
from jax.experimental.pallas import tpu_sc as plsc

_N, _C, _H, _W = 2, 256, 128, 128
_R = 512
_PH = _PW = 7
_S = 2
_SCALE = 0.125
_K = (_S * _S) * 4
_G = (_S * _S) * 2
_BINS = _R * _PH * _PW
_COLS = _PH * _PW * _K
_NC, _NS = 2, 16
_NW = _NC * _NS
_BPW = _BINS // _NW
_OCH = 112
_NOC = _BPW // _OCH
_GRP = 2
_CP = _C // 2
_VPAD = 256
_V = _N * _H * _W + _VPAD


def _col_consts() -> np.ndarray:
    j = np.arange(_COLS)
    b = j // _K
    ph = b // _PW
    pw = b % _PW
    k = j % _K
    s = k // 4
    corner = k % 4
    sy = s // _S
    sx = s % _S
    cyb = (corner // 2).astype(np.float32)
    cxb = (corner % 2).astype(np.float32)
    ay = ((ph + (sy + 0.5) / _S) / _PH - 0.5).astype(np.float32)
    ax = ((pw + (sx + 0.5) / _S) / _PW - 0.5).astype(np.float32)
    return np.stack([ax, ay, cxb, cyb], axis=0)


def _coord_body(rois_ref, cst_ref, idx_ref, wgt_ref):
    b = rois_ref[:, 0:1]
    cx = rois_ref[:, 1:2] * _SCALE
    cy = rois_ref[:, 2:3] * _SCALE
    w = jnp.maximum(rois_ref[:, 3:4] * _SCALE, 1.0)
    h = jnp.maximum(rois_ref[:, 4:5] * _SCALE, 1.0)
    th = rois_ref[:, 5:6]
    cos_t = jnp.cos(th)
    sin_t = jnp.sin(th)
    ax = cst_ref[0:1, :]
    ay = cst_ref[1:2, :]
    cxb = cst_ref[2:3, :]
    cyb = cst_ref[3:4, :]
    xx = w * ax
    yy = h * ay
    x = xx * cos_t - yy * sin_t + cx
    y = xx * sin_t + yy * cos_t + cy
    valid = (y > -1.0) & (y < float(_H)) & (x > -1.0) & (x < float(_W))
    xc = jnp.clip(x, 0.0, float(_W - 1))
    yc = jnp.clip(y, 0.0, float(_H - 1))
    x0 = jnp.floor(xc)
    y0 = jnp.floor(yc)
    lx = xc - x0
    ly = yc - y0
    wx = jnp.where(cxb > 0.5, lx, 1.0 - lx)
    wy = jnp.where(cyb > 0.5, ly, 1.0 - ly)
    wgt_ref[...] = jnp.where(valid, wx * wy * 0.25, 0.0)
    idx_f = (b * float(_H) + y0) * float(_W) + x0 + cxb + cyb * float(_W)
    idx_ref[...] = idx_f.astype(jnp.int32)


_coord_call = pl.pallas_call(
    _coord_body,
    out_shape=(
        jax.ShapeDtypeStruct((_R, _COLS), jnp.int32),
        jax.ShapeDtypeStruct((_R, _COLS), jnp.float32),
    ),
)


def _sc_body(table, idxs, wgts, out, idx_v, wgt_v, rows_v, out_v,
             sem0, sem1, sem2, sem3):
    cid = lax.axis_index("c")
    sid = lax.axis_index("s")
    wid = sid * _NC + cid
    base = wid * _BPW
    pltpu.sync_copy(idxs.at[pl.ds(base * _K, _BPW * _K)], idx_v)
    pltpu.sync_copy(wgts.at[pl.ds(base * _K, _BPW * _K)], wgt_v)
    sems = (sem0, sem1, sem2, sem3)

    def fire(bl, buf):
        pltpu.make_async_copy(
            table.at[idx_v.at[pl.ds(bl * _K, _K)]],
            rows_v.at[buf], sems[buf]).start()

    def wait(buf):
        pltpu.make_async_copy(
            table.at[idx_v.at[pl.ds(0, _K)]],
            rows_v.at[buf], sems[buf]).wait()

    def compute(bl, ob, buf):
        wrow = wgt_v[pl.ds(bl * _K, _K)]
        wks = [
            wrow.at[jnp.full((16,), k, jnp.int32)].get(
                mode="promise_in_bounds")
            for k in range(_K)
        ]
        hi_mask = jnp.full((16,), jnp.int32(-65536))
        for c16 in range(_CP // 16):
            acc_e = acc_o = None
            for k in range(_K):
                v = rows_v[buf, k, pl.ds(c16 * 16, 16)]
                lo = lax.bitcast_convert_type(v << 16, jnp.float32)
                hi = lax.bitcast_convert_type(v & hi_mask, jnp.float32)
                if k == 0:
                    acc_e = lo * wks[0]
                    acc_o = hi * wks[0]
                else:
                    acc_e = acc_e + lo * wks[k]
                    acc_o = acc_o + hi * wks[k]
            out_v[ob, pl.ds(c16 * 16, 16)] = acc_e
            out_v[ob, pl.ds(_CP + c16 * 16, 16)] = acc_o

    for p in range(_GRP - 1):
        fire(p, p)

    def oc_body(oc, carry):
        obase = oc * _OCH

        def grp_body(g, carry2):
            for u in range(_GRP):
                bl = obase + g * _GRP + u
                buf = u
                nxt = bl + _GRP - 1

                @pl.when(nxt < _BPW)
                def _():
                    fire(nxt, (u + _GRP - 1) % _GRP)

                wait(buf)
                compute(bl, g * _GRP + u, buf)
            return carry2

        lax.fori_loop(0, _OCH // _GRP, grp_body, None)
        pltpu.sync_copy(out_v, out.at[pl.ds(base + obase, _OCH)])
        return carry

    lax.fori_loop(0, _NOC, oc_body, None)


@functools.cache
def _sc_gather():
    mesh = plsc.VectorSubcoreMesh(
        core_axis_name="c", subcore_axis_name="s",
        num_cores=_NC, num_subcores=_NS)
    return pl.kernel(
        _sc_body,
        out_type=jax.ShapeDtypeStruct((_BINS, _C), jnp.float32),
        mesh=mesh,
        scratch_types=[
            pltpu.VMEM((_BPW * _K,), jnp.int32),
            pltpu.VMEM((_BPW * _K,), jnp.float32),
            pltpu.VMEM((_GRP, _K, _CP), jnp.int32),
            pltpu.VMEM((_OCH, _C), jnp.float32),
            pltpu.SemaphoreType.DMA,
            pltpu.SemaphoreType.DMA,
            pltpu.SemaphoreType.DMA,
            pltpu.SemaphoreType.DMA,
        ],
    )


def _pack_table(inputs):
    feat = jnp.transpose(inputs, (0, 2, 3, 1)).reshape(_N * _H * _W, _C)
    u16 = lax.bitcast_convert_type(
        feat.astype(jnp.bfloat16), jnp.uint16).astype(jnp.uint32)
    u16 = u16.reshape(_N * _H * _W, _CP, 2)
    packed = lax.bitcast_convert_type(
        u16[:, :, 0] | (u16[:, :, 1] << 16), jnp.int32)
    return jnp.concatenate(
        [packed, jnp.zeros((_VPAD, _CP), jnp.int32)], axis=0)


def kernel(inputs, rois):
    table = _pack_table(inputs)
    consts = jnp.asarray(_col_consts())
    idx, wgt = _coord_call(rois, consts)
    out = _sc_gather()(
        table, idx.reshape(_BINS * _K), wgt.reshape(_BINS * _K))
    out = out.reshape(_BINS, 2, _CP).transpose(0, 2, 1).reshape(_BINS, _C)
    return out.reshape(_R, _PH, _PW, _C).transpose(0, 3, 1, 2)


# speedup vs baseline: 1.0424x; 1.0424x over previous
import functools

import numpy as np
import jax
import jax.numpy as jnp
from jax import lax
from jax.experimental import pallas as pl
from jax.experimental.pallas import tpu as pltpu
from jax.experimental.pallas import tpu_sc as plsc

_N, _C, _H, _W = 2, 256, 128, 128
_R = 512
_PH = _PW = 7
_S = 2
_SCALE = 0.125
_K = (_S * _S) * 4
_G = (_S * _S) * 2
_BINS = _R * _PH * _PW
_COLS = _PH * _PW * _K
_NC, _NS = 2, 16
_NW = _NC * _NS
_BPW = _BINS // _NW
_OCH = 112
_NOC = _BPW // _OCH
_GRP = 2
_CP = _C // 2
_VPAD = 256
_V = _N * _H * _W + _VPAD


def _col_consts() -> np.ndarray:
    j = np.arange(_COLS)
    b = j // _K
    ph = b // _PW
    pw = b % _PW
    k = j % _K
    s = k // 4
    corner = k % 4
    sy = s // _S
    sx = s % _S
    cyb = (corner // 2).astype(np.float32)
    cxb = (corner % 2).astype(np.float32)
    ay = ((ph + (sy + 0.5) / _S) / _PH - 0.5).astype(np.float32)
    ax = ((pw + (sx + 0.5) / _S) / _PW - 0.5).astype(np.float32)
    return np.stack([ax, ay, cxb, cyb], axis=0)


def _coord_body(rois_ref, cst_ref, idx_ref, wgt_ref):
    b = rois_ref[:, 0:1]
    cx = rois_ref[:, 1:2] * _SCALE
    cy = rois_ref[:, 2:3] * _SCALE
    w = jnp.maximum(rois_ref[:, 3:4] * _SCALE, 1.0)
    h = jnp.maximum(rois_ref[:, 4:5] * _SCALE, 1.0)
    th = rois_ref[:, 5:6]
    cos_t = jnp.cos(th)
    sin_t = jnp.sin(th)
    ax = cst_ref[0:1, :]
    ay = cst_ref[1:2, :]
    cxb = cst_ref[2:3, :]
    cyb = cst_ref[3:4, :]
    xx = w * ax
    yy = h * ay
    x = xx * cos_t - yy * sin_t + cx
    y = xx * sin_t + yy * cos_t + cy
    valid = (y > -1.0) & (y < float(_H)) & (x > -1.0) & (x < float(_W))
    xc = jnp.clip(x, 0.0, float(_W - 1))
    yc = jnp.clip(y, 0.0, float(_H - 1))
    x0 = jnp.floor(xc)
    y0 = jnp.floor(yc)
    lx = xc - x0
    ly = yc - y0
    wx = jnp.where(cxb > 0.5, lx, 1.0 - lx)
    wy = jnp.where(cyb > 0.5, ly, 1.0 - ly)
    wgt_ref[...] = jnp.where(valid, wx * wy * 0.25, 0.0)
    idx_f = (b * float(_H) + y0) * float(_W) + x0 + cxb + cyb * float(_W)
    idx_ref[...] = idx_f.astype(jnp.int32)


_coord_call = pl.pallas_call(
    _coord_body,
    out_shape=(
        jax.ShapeDtypeStruct((_R, _COLS), jnp.int32),
        jax.ShapeDtypeStruct((_R, _COLS), jnp.float32),
    ),
)


def _sc_body(table, idxs, wgts, out, idx_v, wgt_v, rows_v, out_v,
             sem0, sem1, sem2, sem3):
    cid = lax.axis_index("c")
    sid = lax.axis_index("s")
    wid = sid * _NC + cid
    base = wid * _BPW
    pltpu.sync_copy(idxs.at[pl.ds(base * _K, _BPW * _K)], idx_v)
    pltpu.sync_copy(wgts.at[pl.ds(base * _K, _BPW * _K)], wgt_v)
    sems = (sem0, sem1, sem2, sem3)

    def fire(bl, buf):
        pltpu.make_async_copy(
            table.at[idx_v.at[pl.ds(bl * _K, _K)]],
            rows_v.at[buf], sems[buf]).start()

    def wait(buf):
        pltpu.make_async_copy(
            table.at[idx_v.at[pl.ds(0, _K)]],
            rows_v.at[buf], sems[buf]).wait()

    def compute(bl, ob, buf):
        wrow = wgt_v[pl.ds(bl * _K, _K)]
        wks = [
            wrow.at[jnp.full((16,), k, jnp.int32)].get(
                mode="promise_in_bounds")
            for k in range(_K)
        ]
        for c16 in range(_CP // 16):
            acc_e = acc_o = None
            for k in range(_K):
                v = rows_v[buf, k, pl.ds(c16 * 16, 16)]
                lo = lax.bitcast_convert_type(v << 16, jnp.float32)
                hi = lax.bitcast_convert_type(v, jnp.float32)
                if k == 0:
                    acc_e = lo * wks[0]
                    acc_o = hi * wks[0]
                else:
                    acc_e = acc_e + lo * wks[k]
                    acc_o = acc_o + hi * wks[k]
            out_v[ob, pl.ds(c16 * 16, 16)] = acc_e
            out_v[ob, pl.ds(_CP + c16 * 16, 16)] = acc_o

    for p in range(_GRP - 1):
        fire(p, p)

    def oc_body(oc, carry):
        obase = oc * _OCH

        def grp_body(g, carry2):
            for u in range(_GRP):
                bl = obase + g * _GRP + u
                buf = u
                nxt = bl + _GRP - 1

                @pl.when(nxt < _BPW)
                def _():
                    fire(nxt, (u + _GRP - 1) % _GRP)

                wait(buf)
                compute(bl, g * _GRP + u, buf)
            return carry2

        lax.fori_loop(0, _OCH // _GRP, grp_body, None)
        pltpu.sync_copy(out_v, out.at[pl.ds(base + obase, _OCH)])
        return carry

    lax.fori_loop(0, _NOC, oc_body, None)


@functools.cache
def _sc_gather():
    mesh = plsc.VectorSubcoreMesh(
        core_axis_name="c", subcore_axis_name="s",
        num_cores=_NC, num_subcores=_NS)
    return pl.kernel(
        _sc_body,
        out_type=jax.ShapeDtypeStruct((_BINS, _C), jnp.float32),
        mesh=mesh,
        scratch_types=[
            pltpu.VMEM((_BPW * _K,), jnp.int32),
            pltpu.VMEM((_BPW * _K,), jnp.float32),
            pltpu.VMEM((_GRP, _K, _CP), jnp.int32),
            pltpu.VMEM((_OCH, _C), jnp.float32),
            pltpu.SemaphoreType.DMA,
            pltpu.SemaphoreType.DMA,
            pltpu.SemaphoreType.DMA,
            pltpu.SemaphoreType.DMA,
        ],
    )


def _pack_table(inputs):
    feat = jnp.transpose(inputs, (0, 2, 3, 1)).reshape(_N * _H * _W, _C)
    packed = lax.bitcast_convert_type(
        feat.astype(jnp.bfloat16).reshape(_N * _H * _W, _CP, 2), jnp.int32)
    return jnp.concatenate(
        [packed, jnp.zeros((_VPAD, _CP), jnp.int32)], axis=0)


def kernel(inputs, rois):
    table = _pack_table(inputs)
    consts = jnp.asarray(_col_consts())
    idx, wgt = _coord_call(rois, consts)
    out = _sc_gather()(
        table, idx.reshape(_BINS * _K), wgt.reshape(_BINS * _K))
    out = out.reshape(_BINS, 2, _CP).transpose(0, 2, 1).reshape(_BINS, _C)
    return out.reshape(_R, _PH, _PW, _C).transpose(0, 3, 1, 2)


# speedup vs baseline: 1.8735x; 1.7973x over previous
import functools

import numpy as np
import jax
import jax.numpy as jnp
from jax import lax
from jax.experimental import pallas as pl
from jax.experimental.pallas import tpu as pltpu
from jax.experimental.pallas import tpu_sc as plsc

_N, _C, _H, _W = 2, 256, 128, 128
_R = 512
_PH = _PW = 7
_S = 2
_SCALE = 0.125
_K = (_S * _S) * 4
_BINS = _R * _PH * _PW
_COLS = _PH * _PW * _K
_NC, _NS = 2, 16
_NW = _NC * _NS
_BPW = _BINS // _NW
_OCH = 112
_NOC = _BPW // _OCH
_GRP = 2
_VPAD = 256
_V = _N * _H * _W + _VPAD


def _col_consts() -> np.ndarray:
    j = np.arange(_COLS)
    b = j // _K
    ph = b // _PW
    pw = b % _PW
    k = j % _K
    s = k // 4
    corner = k % 4
    sy = s // _S
    sx = s % _S
    cyb = (corner // 2).astype(np.float32)
    cxb = (corner % 2).astype(np.float32)
    ay = ((ph + (sy + 0.5) / _S) / _PH - 0.5).astype(np.float32)
    ax = ((pw + (sx + 0.5) / _S) / _PW - 0.5).astype(np.float32)
    return np.stack([ax, ay, cxb, cyb], axis=0)


def _coord_body(rois_ref, cst_ref, idx_ref, wgt_ref):
    b = rois_ref[:, 0:1]
    cx = rois_ref[:, 1:2] * _SCALE
    cy = rois_ref[:, 2:3] * _SCALE
    w = jnp.maximum(rois_ref[:, 3:4] * _SCALE, 1.0)
    h = jnp.maximum(rois_ref[:, 4:5] * _SCALE, 1.0)
    th = rois_ref[:, 5:6]
    cos_t = jnp.cos(th)
    sin_t = jnp.sin(th)
    ax = cst_ref[0:1, :]
    ay = cst_ref[1:2, :]
    cxb = cst_ref[2:3, :]
    cyb = cst_ref[3:4, :]
    xx = w * ax
    yy = h * ay
    x = xx * cos_t - yy * sin_t + cx
    y = xx * sin_t + yy * cos_t + cy
    valid = (y > -1.0) & (y < float(_H)) & (x > -1.0) & (x < float(_W))
    xc = jnp.clip(x, 0.0, float(_W - 1))
    yc = jnp.clip(y, 0.0, float(_H - 1))
    x0 = jnp.floor(xc)
    y0 = jnp.floor(yc)
    lx = xc - x0
    ly = yc - y0
    wx = jnp.where(cxb > 0.5, lx, 1.0 - lx)
    wy = jnp.where(cyb > 0.5, ly, 1.0 - ly)
    wgt_ref[...] = jnp.where(valid, wx * wy * 0.25, 0.0)
    idx_f = (b * float(_H) + y0) * float(_W) + x0 + cxb + cyb * float(_W)
    idx_ref[...] = idx_f.astype(jnp.int32)


_coord_call = pl.pallas_call(
    _coord_body,
    out_shape=(
        jax.ShapeDtypeStruct((_R, _COLS), jnp.int32),
        jax.ShapeDtypeStruct((_R, _COLS), jnp.float32),
    ),
)


def _sc_body(table, idxs, wgts, out, idx_v, wgt_v, rows_v, out_v, sem0, sem1):
    cid = lax.axis_index("c")
    sid = lax.axis_index("s")
    wid = sid * _NC + cid
    base = wid * _BPW
    pltpu.sync_copy(idxs.at[pl.ds(base * _K, _BPW * _K)], idx_v)
    pltpu.sync_copy(wgts.at[pl.ds(base * _K, _BPW * _K)], wgt_v)
    sems = (sem0, sem1)

    def fire(bl, buf):
        pltpu.make_async_copy(
            table.at[idx_v.at[pl.ds(bl * _K, _K)]],
            rows_v.at[buf], sems[buf]).start()

    def wait(buf):
        pltpu.make_async_copy(
            table.at[idx_v.at[pl.ds(0, _K)]],
            rows_v.at[buf], sems[buf]).wait()

    def compute(bl, ob, buf):
        wrow = wgt_v[pl.ds(bl * _K, _K)]
        wks = [
            wrow.at[jnp.full((16,), k, jnp.int32)].get(
                mode="promise_in_bounds")
            for k in range(_K)
        ]
        for c16 in range(_C // 16):
            acc = rows_v[buf, 0, pl.ds(c16 * 16, 16)] * wks[0]
            for k in range(1, _K):
                acc = acc + rows_v[buf, k, pl.ds(c16 * 16, 16)] * wks[k]
            out_v[ob, pl.ds(c16 * 16, 16)] = acc

    fire(0, 0)

    def oc_body(oc, carry):
        obase = oc * _OCH

        def grp_body(g, carry2):
            for u in range(_GRP):
                bl = obase + g * _GRP + u
                buf = u
                nxt = bl + 1

                @pl.when(nxt < _BPW)
                def _():
                    fire(nxt, (u + 1) % _GRP)

                wait(buf)
                compute(bl, g * _GRP + u, buf)
            return carry2

        lax.fori_loop(0, _OCH // _GRP, grp_body, None)
        pltpu.sync_copy(out_v, out.at[pl.ds(base + obase, _OCH)])
        return carry

    lax.fori_loop(0, _NOC, oc_body, None)


@functools.cache
def _sc_gather():
    mesh = plsc.VectorSubcoreMesh(
        core_axis_name="c", subcore_axis_name="s",
        num_cores=_NC, num_subcores=_NS)
    return pl.kernel(
        _sc_body,
        out_type=jax.ShapeDtypeStruct((_BINS, _C), jnp.float32),
        mesh=mesh,
        scratch_types=[
            pltpu.VMEM((_BPW * _K,), jnp.int32),
            pltpu.VMEM((_BPW * _K,), jnp.float32),
            pltpu.VMEM((_GRP, _K, _C), jnp.float32),
            pltpu.VMEM((_OCH, _C), jnp.float32),
            pltpu.SemaphoreType.DMA,
            pltpu.SemaphoreType.DMA,
        ],
    )


def kernel(inputs, rois):
    feat = jnp.transpose(inputs, (0, 2, 3, 1)).reshape(_N * _H * _W, _C)
    table = jnp.concatenate(
        [feat, jnp.zeros((_VPAD, _C), jnp.float32)], axis=0)
    consts = jnp.asarray(_col_consts())
    idx, wgt = _coord_call(rois, consts)
    out = _sc_gather()(
        table, idx.reshape(_BINS * _K), wgt.reshape(_BINS * _K))
    return out.reshape(_R, _PH, _PW, _C).transpose(0, 3, 1, 2)
